# baseline (speedup 1.0000x reference)
import functools

import jax
import jax.numpy as jnp
import numpy as np
from jax import lax
from jax.experimental import pallas as pl
from jax.experimental.pallas import tpu as pltpu
from jax.experimental.pallas import tpu_sc as plsc

_VOCAB = 100000
_D = 64
_SEQ = 200
_BATCH = 4096
_NC, _NS, _L = 2, 16, 16
_NW = _NC * _NS
_BB = _BATCH // _NW
_NBUF = 4
_NSTG = 2
_LEAD = 2
_NG = _SEQ // _NBUF


def _pe_table() -> np.ndarray:
    position = np.arange(_SEQ, dtype=np.float32)[:, None]
    div = np.exp(np.arange(0, _D, 2, dtype=np.float32) * -(np.log(10000.0) / _D))
    pe = np.zeros((_SEQ, _D), np.float32)
    pe[:, 0::2] = np.sin(position * div)
    pe[:, 1::2] = np.cos(position * div)
    return pe.reshape(-1)


_PE = _pe_table()


@functools.partial(
    pl.kernel,
    mesh=plsc.VectorSubcoreMesh(core_axis_name="c", subcore_axis_name="s"),
    out_type=jax.ShapeDtypeStruct((_SEQ, 8, _NW, 8, _BB), jnp.float32),
    scratch_types=[
        pltpu.VMEM((_SEQ, _BB), jnp.int32),
        pltpu.VMEM((_NBUF, _BB, _D), jnp.float32),
        pltpu.VMEM((_NSTG, 8, 8, _BB + 1), jnp.float32),
        pltpu.VMEM((_SEQ * _D,), jnp.float32),
        pltpu.SemaphoreType.DMA((_NBUF,)),
        pltpu.SemaphoreType.DMA((_NSTG,)),
    ],
    compiler_params=pltpu.CompilerParams(
        use_tc_tiling_on_sc=False, needs_layout_passes=False),
)
def _emb_kernel(xt_hbm, w_hbm, pe_hbm, out_hbm, idx_v, rows_v, stg_v, pe_v,
                sem_g, sem_o):
    wid = lax.axis_index("s") * _NC + lax.axis_index("c")

    pltpu.sync_copy(xt_hbm.at[:, pl.ds(wid * _BB, _BB)], idx_v)
    pltpu.sync_copy(pe_hbm, pe_v)

    def start_gather(s, slot):
        pltpu.async_copy(w_hbm.at[idx_v.at[s]], rows_v.at[slot], sem_g.at[slot])

    def wait_gather(s, slot):
        pltpu.make_async_copy(
            w_hbm.at[idx_v.at[s]], rows_v.at[slot], sem_g.at[slot]).wait()

    def start_out(s, stg):
        pltpu.async_copy(
            stg_v.at[stg, slice(None), slice(None), pl.ds(0, _BB)],
            out_hbm.at[s, slice(None), wid], sem_o.at[stg])

    def wait_out(s, stg):
        pltpu.make_async_copy(
            stg_v.at[stg, slice(None), slice(None), pl.ds(0, _BB)],
            out_hbm.at[s, slice(None), wid], sem_o.at[stg]).wait()

    def transpose_add(s, slot, stg):
        lanes = lax.iota(jnp.int32, _L)
        dhis = [lanes // 8 + 2 * k for k in range(_D // _L)]
        dlo = lanes % 8
        pe_regs = [pe_v[pl.ds(s * _D + k * _L, _L)] for k in range(_D // _L)]

        def i_body(i, acc):
            for u in range(2):
                ii = i * 2 + u
                li = jnp.full((_L,), ii, jnp.int32)
                for k in range(_D // _L):
                    v = rows_v[slot, ii, pl.ds(k * _L, _L)]
                    plsc.store_scatter(
                        stg_v.at[stg], [dhis[k], dlo, li], v + pe_regs[k])
            return acc

        lax.fori_loop(0, _BB // 2, i_body, 0)

    def emit(s, b, do_wait_out, do_prefetch):
        stg = b % _NSTG
        if do_prefetch:
            start_gather(s + _LEAD, (b + _LEAD) % _NBUF)
        wait_gather(s, b)
        if do_wait_out:
            wait_out(s - _NSTG, stg)
        transpose_add(s, b, stg)
        start_out(s, stg)

    for s0 in range(_LEAD):
        start_gather(s0, s0 % _NBUF)

    for b in range(_NBUF):
        emit(b, b, do_wait_out=(b >= _NSTG), do_prefetch=True)

    def group_body(g, acc):
        s0 = g * _NBUF
        for b in range(_NBUF):
            emit(s0 + b, b, do_wait_out=True, do_prefetch=True)
        return acc

    lax.fori_loop(1, _NG - 1, group_body, 0)

    sl = (_NG - 1) * _NBUF
    for b in range(_NBUF):
        emit(sl + b, b, do_wait_out=True,
             do_prefetch=(sl + b + _LEAD < _SEQ))

    for b in range(_NSTG):
        wait_out(sl + 2 + b, b)


def kernel(x, W):
    xt = x.T.astype(jnp.int32)
    out5 = _emb_kernel(xt, W, jnp.asarray(_PE))
    return out5.transpose(2, 4, 0, 1, 3).reshape(_BATCH, _SEQ, _D)
